# baseline (speedup 1.0000x reference)
import functools

import jax
import jax.numpy as jnp
from jax import lax
from jax.experimental import pallas as pl
from jax.experimental.pallas import tpu as pltpu
from jax.experimental.pallas import tpu_sc as plsc

_CHUNK = 128


@functools.cache
def _build(batch: int, n_states: int):
  info = plsc.get_sparse_core_info()
  nw = info.num_cores * info.num_subcores
  rows = batch // _CHUNK
  rows_per_w = rows // nw

  mesh = plsc.VectorSubcoreMesh(core_axis_name="c", subcore_axis_name="s")

  @functools.partial(
      pl.kernel,
      mesh=mesh,
      out_type=jax.ShapeDtypeStruct((rows * _CHUNK,), jnp.float32),
      scratch_types=[
          pltpu.VMEM((rows_per_w * _CHUNK // 2,), jnp.int32),
          pltpu.VMEM((rows_per_w * _CHUNK // 2,), jnp.int32),
          pltpu.VMEM((rows_per_w * _CHUNK // 2,), jnp.float32),
          pltpu.VMEM((rows_per_w * _CHUNK // 2,), jnp.float32),
          pltpu.SemaphoreType.DMA,
          pltpu.SemaphoreType.DMA,
          pltpu.SemaphoreType.DMA,
          pltpu.SemaphoreType.DMA,
          pltpu.SemaphoreType.DMA,
          pltpu.SemaphoreType.DMA,
      ],
  )
  def gather_kernel(state_hbm, value_hbm, out_hbm, idx0, idx1, val0, val1,
                    si0, si1, sg0, sg1, sw0, sw1):
    wid = lax.axis_index("s") * info.num_cores + lax.axis_index("c")
    half = rows_per_w * _CHUNK // 2
    base = wid * 2 * half
    i0 = pltpu.async_copy(state_hbm.at[pl.ds(base, half)], idx0, si0)
    i1 = pltpu.async_copy(state_hbm.at[pl.ds(base + half, half)], idx1, si1)
    i0.wait()
    g0 = pltpu.async_copy(value_hbm.at[idx0], val0, sg0)
    i1.wait()
    g1 = pltpu.async_copy(value_hbm.at[idx1], val1, sg1)
    g0.wait()
    w0 = pltpu.async_copy(val0, out_hbm.at[pl.ds(base, half)], sw0)
    g1.wait()
    w1 = pltpu.async_copy(val1, out_hbm.at[pl.ds(base + half, half)], sw1)
    w0.wait()
    w1.wait()

  return gather_kernel


def kernel(state, value):
  batch = state.shape[0]
  return _build(batch, value.shape[0])(state.astype(jnp.int32), value)


# speedup vs baseline: 1.0013x; 1.0013x over previous
import functools

import jax
import jax.numpy as jnp
from jax import lax
from jax.experimental import pallas as pl
from jax.experimental.pallas import tpu as pltpu
from jax.experimental.pallas import tpu_sc as plsc

_CHUNK = 128


@functools.cache
def _build(batch: int, n_states: int):
  info = plsc.get_sparse_core_info()
  nw = info.num_cores * info.num_subcores
  rows = batch // _CHUNK
  rows_per_w = rows // nw

  mesh = plsc.VectorSubcoreMesh(core_axis_name="c", subcore_axis_name="s")

  @functools.partial(
      pl.kernel,
      mesh=mesh,
      out_type=jax.ShapeDtypeStruct((rows * _CHUNK,), jnp.float32),
      scratch_types=[
          pltpu.VMEM((rows_per_w * _CHUNK,), jnp.int32),
          pltpu.VMEM((rows_per_w * _CHUNK,), jnp.float32),
          pltpu.SemaphoreType.DMA,
      ],
  )
  def gather_kernel(state_hbm, value_hbm, out_hbm, idx_v, vals_v, sem_g):
    wid = lax.axis_index("s") * info.num_cores + lax.axis_index("c")
    n_per_w = rows_per_w * _CHUNK
    base = wid * n_per_w
    pltpu.sync_copy(state_hbm.at[pl.ds(base, n_per_w)], idx_v)
    pltpu.async_copy(value_hbm.at[idx_v], vals_v, sem_g).wait()
    pltpu.sync_copy(vals_v, out_hbm.at[pl.ds(base, n_per_w)])

  return gather_kernel


def kernel(state, value):
  batch = state.shape[0]
  return _build(batch, value.shape[0])(state.astype(jnp.int32), value)


# speedup vs baseline: 1.0369x; 1.0356x over previous
import functools

import jax
import jax.numpy as jnp
from jax import lax
from jax.experimental import pallas as pl
from jax.experimental.pallas import tpu as pltpu
from jax.experimental.pallas import tpu_sc as plsc

_CHUNK = 128


@functools.cache
def _build(batch: int, n_states: int):
  info = plsc.get_sparse_core_info()
  nw = 1 * info.num_subcores
  rows = batch // _CHUNK
  rows_per_w = rows // nw

  mesh = plsc.VectorSubcoreMesh(core_axis_name="c", subcore_axis_name="s", num_cores=1)

  @functools.partial(
      pl.kernel,
      mesh=mesh,
      out_type=jax.ShapeDtypeStruct((rows * _CHUNK,), jnp.float32),
      scratch_types=[
          pltpu.VMEM((rows_per_w * _CHUNK,), jnp.int32),
          pltpu.VMEM((rows_per_w * _CHUNK,), jnp.float32),
          pltpu.SemaphoreType.DMA,
      ],
  )
  def gather_kernel(state_hbm, value_hbm, out_hbm, idx_v, vals_v, sem_g):
    wid = lax.axis_index("s")
    n_per_w = rows_per_w * _CHUNK
    base = wid * n_per_w
    pltpu.sync_copy(state_hbm.at[pl.ds(base, n_per_w)], idx_v)
    pltpu.async_copy(value_hbm.at[idx_v], vals_v, sem_g).wait()
    pltpu.sync_copy(vals_v, out_hbm.at[pl.ds(base, n_per_w)])

  return gather_kernel


def kernel(state, value):
  batch = state.shape[0]
  return _build(batch, value.shape[0])(state.astype(jnp.int32), value)


# speedup vs baseline: 1.0500x; 1.0126x over previous
import functools

import jax
import jax.numpy as jnp
from jax import lax
from jax.experimental import pallas as pl
from jax.experimental.pallas import tpu as pltpu
from jax.experimental.pallas import tpu_sc as plsc


@functools.cache
def _build(batch: int, n_states: int):
  info = plsc.get_sparse_core_info()
  nw = info.num_subcores
  n_per_w = batch // nw
  half = n_per_w // 2

  mesh = plsc.VectorSubcoreMesh(
      core_axis_name="c", subcore_axis_name="s", num_cores=1)

  @functools.partial(
      pl.kernel,
      mesh=mesh,
      out_type=jax.ShapeDtypeStruct((batch,), jnp.float32),
      scratch_types=[
          pltpu.VMEM((half,), jnp.int32),
          pltpu.VMEM((half,), jnp.int32),
          pltpu.VMEM((half,), jnp.float32),
          pltpu.VMEM((half,), jnp.float32),
          pltpu.SemaphoreType.DMA,
          pltpu.SemaphoreType.DMA,
          pltpu.SemaphoreType.DMA,
          pltpu.SemaphoreType.DMA,
          pltpu.SemaphoreType.DMA,
          pltpu.SemaphoreType.DMA,
      ],
  )
  def gather_kernel(state_hbm, value_hbm, out_hbm, idx0, idx1, val0, val1,
                    si0, si1, sg0, sg1, sw0, sw1):
    base = lax.axis_index("s") * n_per_w
    i0 = pltpu.async_copy(state_hbm.at[pl.ds(base, half)], idx0, si0)
    i1 = pltpu.async_copy(state_hbm.at[pl.ds(base + half, half)], idx1, si1)
    i0.wait()
    g0 = pltpu.async_copy(value_hbm.at[idx0], val0, sg0)
    i1.wait()
    g1 = pltpu.async_copy(value_hbm.at[idx1], val1, sg1)
    g0.wait()
    w0 = pltpu.async_copy(val0, out_hbm.at[pl.ds(base, half)], sw0)
    g1.wait()
    w1 = pltpu.async_copy(val1, out_hbm.at[pl.ds(base + half, half)], sw1)
    w0.wait()
    w1.wait()

  return gather_kernel


def kernel(state, value):
  batch = state.shape[0]
  return _build(batch, value.shape[0])(state.astype(jnp.int32), value)


# speedup vs baseline: 1.0536x; 1.0034x over previous
import functools

import jax
import jax.numpy as jnp
from jax import lax
from jax.experimental import pallas as pl
from jax.experimental.pallas import tpu as pltpu
from jax.experimental.pallas import tpu_sc as plsc


@functools.cache
def _build(batch: int, n_states: int):
  info = plsc.get_sparse_core_info()
  nw = info.num_subcores
  n_per_w = batch // nw
  n_chunks = 4
  chunk = n_per_w // n_chunks

  mesh = plsc.VectorSubcoreMesh(
      core_axis_name="c", subcore_axis_name="s", num_cores=1)

  scratch = (
      [pltpu.VMEM((chunk,), jnp.int32) for _ in range(n_chunks)]
      + [pltpu.VMEM((chunk,), jnp.float32) for _ in range(n_chunks)]
      + [pltpu.SemaphoreType.DMA for _ in range(3 * n_chunks)]
  )

  @functools.partial(
      pl.kernel,
      mesh=mesh,
      out_type=jax.ShapeDtypeStruct((batch,), jnp.float32),
      scratch_types=scratch,
  )
  def gather_kernel(state_hbm, value_hbm, out_hbm, *refs):
    idx = refs[:n_chunks]
    val = refs[n_chunks:2 * n_chunks]
    si = refs[2 * n_chunks:3 * n_chunks]
    sg = refs[3 * n_chunks:4 * n_chunks]
    sw = refs[4 * n_chunks:5 * n_chunks]
    base = lax.axis_index("s") * n_per_w
    loads = [
        pltpu.async_copy(state_hbm.at[pl.ds(base + j * chunk, chunk)], idx[j],
                         si[j]) for j in range(n_chunks)
    ]
    gathers = []
    for j in range(n_chunks):
      loads[j].wait()
      gathers.append(
          pltpu.async_copy(value_hbm.at[idx[j]], val[j], sg[j]))
    writes = []
    for j in range(n_chunks):
      gathers[j].wait()
      writes.append(
          pltpu.async_copy(val[j], out_hbm.at[pl.ds(base + j * chunk, chunk)],
                           sw[j]))
    for w in writes:
      w.wait()

  return gather_kernel


def kernel(state, value):
  batch = state.shape[0]
  return _build(batch, value.shape[0])(state.astype(jnp.int32), value)
